# baseline (speedup 1.0000x reference)
import functools

import jax
import jax.numpy as jnp
from jax import lax
from jax.experimental import pallas as pl
from jax.experimental.pallas import tpu as pltpu
from jax.experimental.pallas import tpu_sc as plsc

M, DM, DFF, E, TOPK = 2048, 2048, 2048, 8, 2
DH = DM // 2
B = 512
MAXNB = (TOPK * M) // B + E
P = MAXNB * B
NW = 32
TPW = (TOPK * M) // NW
CH = 32


def _pack_pair(lo_f32, hi_f32):
    lob = lax.shift_right_logical(
        lax.bitcast_convert_type(
            lo_f32.astype(jnp.bfloat16).astype(jnp.float32), jnp.int32), 16)
    hib = jnp.bitwise_and(
        lax.bitcast_convert_type(
            hi_f32.astype(jnp.bfloat16).astype(jnp.float32), jnp.int32),
        jnp.int32(-65536))
    return jnp.bitwise_or(hib, lob)


def _unpack_lo(xi):
    return lax.bitcast_convert_type(
        jnp.left_shift(xi, 16), jnp.float32).astype(jnp.bfloat16)


def _unpack_hi(xi):
    return lax.bitcast_convert_type(
        jnp.bitwise_and(xi, jnp.int32(-65536)),
        jnp.float32).astype(jnp.bfloat16)


def _meta_body(ids_ref, dest_ref, be_ref):
    ids = ids_ref[...]
    col = lax.broadcasted_iota(jnp.int32, (M, M), 1)
    row = lax.broadcasted_iota(jnp.int32, (M, M), 0)
    tri = (row <= col).astype(jnp.float32)
    occs, cexcls, counts = [], [], []
    for e in range(E):
        occ = (ids == e).astype(jnp.float32)
        c = lax.dot_general(occ, tri, (((1,), (0,)), ((), ())),
                            preferred_element_type=jnp.float32)
        c0_tot = c[0:1, M - 1:M]
        c = c + jnp.concatenate(
            [jnp.zeros((1, M), jnp.float32),
             jnp.broadcast_to(c0_tot, (1, M))], axis=0)
        occs.append(occ)
        cexcls.append(c - occ)
        counts.append(c[1:2, M - 1:M])
    dest = jnp.zeros((TOPK, M), jnp.float32)
    run = jnp.zeros((1, 1), jnp.float32)
    cumnb = []
    for e in range(E):
        dest = dest + occs[e] * (run * B + cexcls[e])
        nb = jnp.floor((counts[e] + (B - 1)) / B)
        run = run + nb
        cumnb.append(run)
    dest_ref[...] = dest.astype(jnp.int32)
    bidx = lax.broadcasted_iota(jnp.int32, (1, MAXNB), 1).astype(jnp.float32)
    be = jnp.zeros((1, MAXNB), jnp.float32)
    for e in range(E):
        be = be + (bidx >= jnp.broadcast_to(cumnb[e], (1, MAXNB))).astype(
            jnp.float32)
    be_ref[...] = be.astype(jnp.int32)


_meta = pl.pallas_call(
    _meta_body,
    out_shape=(jax.ShapeDtypeStruct((TOPK, M), jnp.int32),
               jax.ShapeDtypeStruct((1, MAXNB), jnp.int32)),
)


def _pack_body(a_ref, o_ref):
    a = a_ref[...]
    o_ref[...] = _pack_pair(a[:, :DH], a[:, DH:])


_pack = pl.pallas_call(
    _pack_body,
    grid=(M // B,),
    in_specs=[pl.BlockSpec((B, DM), lambda m: (m, 0))],
    out_specs=pl.BlockSpec((B, DH), lambda m: (m, 0)),
    out_shape=jax.ShapeDtypeStruct((M, DH), jnp.int32),
)


@functools.cache
def _sc_kernels():
    mesh = plsc.VectorSubcoreMesh(core_axis_name="c", subcore_axis_name="s")

    @functools.partial(
        pl.kernel,
        mesh=mesh,
        out_type=jax.ShapeDtypeStruct((P, DH), jnp.int32),
        scratch_types=[
            pltpu.VMEM((2, CH), jnp.int32),
            pltpu.VMEM((2, CH, DH), jnp.int32),
            pltpu.SemaphoreType.DMA((4,)),
        ],
    )
    def dispatch(ap_hbm, destf_hbm, xp_hbm, idx_v, rows_v, sems):
        wid = lax.axis_index("s") * 2 + lax.axis_index("c")
        k = wid // (NW // TOPK)
        t0 = (wid % (NW // TOPK)) * TPW
        nch = TPW // CH
        rd = [None] * nch
        wr = [None] * nch

        def read(ch, buf):
            base = t0 + ch * CH
            pltpu.sync_copy(destf_hbm.at[pl.ds(k * M + base, CH)],
                            idx_v.at[buf])
            return pltpu.async_copy(ap_hbm.at[pl.ds(base, CH), :],
                                    rows_v.at[buf], sems.at[buf])

        rd[0] = read(0, 0)
        for ch in range(nch):
            cur = ch % 2
            nxt = (ch + 1) % 2
            if ch + 1 < nch:
                if wr[ch - 1] is not None:
                    wr[ch - 1].wait()
                rd[ch + 1] = read(ch + 1, nxt)
            rd[ch].wait()
            wr[ch] = pltpu.async_copy(rows_v.at[cur],
                                      xp_hbm.at[idx_v.at[cur]],
                                      sems.at[2 + cur])
        wr[nch - 2].wait()
        wr[nch - 1].wait()

    @functools.partial(
        pl.kernel,
        mesh=mesh,
        out_type=jax.ShapeDtypeStruct((TOPK * M, DH), jnp.int32),
        scratch_types=[
            pltpu.VMEM((2, CH), jnp.int32),
            pltpu.VMEM((2, CH, DH), jnp.int32),
            pltpu.SemaphoreType.DMA((4,)),
        ],
    )
    def gather2(y2_hbm, destf_hbm, g_hbm, idx_v, rows_v, sems):
        wid = lax.axis_index("s") * 2 + lax.axis_index("c")
        k = wid // (NW // TOPK)
        t0 = (wid % (NW // TOPK)) * TPW
        nch = TPW // CH
        rd = [None] * nch
        wr = [None] * nch

        def gat(ch, buf):
            base = k * M + t0 + ch * CH
            pltpu.sync_copy(destf_hbm.at[pl.ds(base, CH)], idx_v.at[buf])
            return pltpu.async_copy(y2_hbm.at[idx_v.at[buf]],
                                    rows_v.at[buf], sems.at[buf])

        rd[0] = gat(0, 0)
        for ch in range(nch):
            cur = ch % 2
            nxt = (ch + 1) % 2
            if ch + 1 < nch:
                if wr[ch - 1] is not None:
                    wr[ch - 1].wait()
                rd[ch + 1] = gat(ch + 1, nxt)
            rd[ch].wait()
            base = k * M + t0 + ch * CH
            wr[ch] = pltpu.async_copy(rows_v.at[cur],
                                      g_hbm.at[pl.ds(base, CH), :],
                                      sems.at[2 + cur])
        wr[nch - 2].wait()
        wr[nch - 1].wait()

    return dispatch, gather2


NS1 = 2
BN1 = DFF // NS1


def _mm1_body(be_ref, xp_ref, wgk0_ref, wgk1_ref, wuk0_ref, wuk1_ref,
              act_ref):
    b = pl.program_id(1)

    @pl.when(be_ref[b] < E)
    def _():
        xi = xp_ref[...]
        xlo = _unpack_lo(xi)
        xhi = _unpack_hi(xi)
        dn = (((1,), (1,)), ((), ()))
        hg = (lax.dot_general(xlo, wgk0_ref[0].astype(jnp.bfloat16), dn,
                              preferred_element_type=jnp.float32)
              + lax.dot_general(xhi, wgk1_ref[0].astype(jnp.bfloat16), dn,
                                preferred_element_type=jnp.float32))
        hu = (lax.dot_general(xlo, wuk0_ref[0].astype(jnp.bfloat16), dn,
                              preferred_element_type=jnp.float32)
              + lax.dot_general(xhi, wuk1_ref[0].astype(jnp.bfloat16), dn,
                                preferred_element_type=jnp.float32))
        actf = hg * jax.nn.sigmoid(hg) * hu
        act_ref[...] = _pack_pair(actf[:, :BN1 // 2], actf[:, BN1 // 2:])


def _mm1_gridspec():
    return pltpu.PrefetchScalarGridSpec(
        num_scalar_prefetch=1,
        grid=(NS1, MAXNB),
        in_specs=[
            pl.BlockSpec((B, DH),
                         lambda n, b, be: (jnp.where(be[b] < E, b, 0), 0)),
            pl.BlockSpec((1, BN1, DH),
                         lambda n, b, be: (jnp.minimum(be[b], E - 1), n, 0)),
            pl.BlockSpec((1, BN1, DH),
                         lambda n, b, be: (jnp.minimum(be[b], E - 1), n, 1)),
            pl.BlockSpec((1, BN1, DH),
                         lambda n, b, be: (jnp.minimum(be[b], E - 1), n + NS1,
                                           0)),
            pl.BlockSpec((1, BN1, DH),
                         lambda n, b, be: (jnp.minimum(be[b], E - 1), n + NS1,
                                           1)),
        ],
        out_specs=pl.BlockSpec(
            (B, BN1 // 2),
            lambda n, b, be: (jnp.where(be[b] < E, b, MAXNB - 1), n)),
    )


_mm1 = pl.pallas_call(
    _mm1_body,
    grid_spec=_mm1_gridspec(),
    out_shape=jax.ShapeDtypeStruct((P, DFF // 2), jnp.int32),
)


def _mm2_body(be_ref, act_ref, w0_ref, w1_ref, w2_ref, w3_ref, y2_ref):
    b = pl.program_id(0)
    Q = BN1 // 2

    @pl.when(be_ref[b] < E)
    def _():
        ai = act_ref[...]
        dn = (((1,), (1,)), ((), ()))
        ws = [w0_ref, w1_ref, w2_ref, w3_ref]
        y = None
        for n in range(NS1):
            blk = ai[:, n * Q:(n + 1) * Q]
            for half, unpack in ((0, _unpack_lo), (1, _unpack_hi)):
                w = ws[2 * n + half][0].astype(jnp.bfloat16)
                part = lax.dot_general(unpack(blk), w, dn,
                                       preferred_element_type=jnp.float32)
                y = part if y is None else y + part
        y2_ref[...] = _pack_pair(y[:, :DH], y[:, DH:])


def _mm2_gridspec():
    return pltpu.PrefetchScalarGridSpec(
        num_scalar_prefetch=1,
        grid=(MAXNB,),
        in_specs=[
            pl.BlockSpec((B, DFF // 2),
                         lambda b, be: (jnp.where(be[b] < E, b, 0), 0)),
        ] + [
            pl.BlockSpec((1, DM, DFF // 4),
                         lambda b, be, q=q: (jnp.minimum(be[b], E - 1), 0, q))
            for q in range(4)
        ],
        out_specs=pl.BlockSpec(
            (B, DH), lambda b, be: (jnp.where(be[b] < E, b, MAXNB - 1), 0)),
    )


_mm2 = pl.pallas_call(
    _mm2_body,
    grid_spec=_mm2_gridspec(),
    out_shape=jax.ShapeDtypeStruct((P, DH), jnp.int32),
)


def _comb_body(g_ref, tw_ref, out_ref):
    g = g_ref[...]
    tw = tw_ref[...]
    w0 = tw[:, 0:1]
    w1 = tw[:, 1:2]
    lo = (_unpack_lo(g[0]).astype(jnp.float32) * w0
          + _unpack_lo(g[1]).astype(jnp.float32) * w1)
    hi = (_unpack_hi(g[0]).astype(jnp.float32) * w0
          + _unpack_hi(g[1]).astype(jnp.float32) * w1)
    out_ref[...] = jnp.concatenate([lo, hi], axis=1)


_BM = 256
_comb = pl.pallas_call(
    _comb_body,
    grid=(M // _BM,),
    in_specs=[
        pl.BlockSpec((TOPK, _BM, DH), lambda m: (0, m, 0)),
        pl.BlockSpec((_BM, TOPK), lambda m: (m, 0)),
    ],
    out_specs=pl.BlockSpec((_BM, DM), lambda m: (m, 0)),
    out_shape=jax.ShapeDtypeStruct((M, DM), jnp.float32),
)


def kernel(a1, w1, w2, topk_weights, topk_ids):
    dispatch, gather2 = _sc_kernels()
    dest, be = _meta(topk_ids.T)
    destf = dest.reshape(TOPK * M)
    be_arr = be.reshape(MAXNB)
    ap = _pack(a1)
    xp = dispatch(ap, destf)
    act = _mm1(be_arr, xp, w1, w1, w1, w1)
    y2 = _mm2(be_arr, act, w2, w2, w2, w2)
    g = gather2(y2, destf)
    return _comb(g.reshape(TOPK, M, DH), topk_weights)


# speedup vs baseline: 1.0022x; 1.0022x over previous
import functools

import jax
import jax.numpy as jnp
from jax import lax
from jax.experimental import pallas as pl
from jax.experimental.pallas import tpu as pltpu
from jax.experimental.pallas import tpu_sc as plsc

M, DM, DFF, E, TOPK = 2048, 2048, 2048, 8, 2
DH = DM // 2
B = 512
MAXNB = (TOPK * M) // B + E
P = MAXNB * B
NW = 32
TPW = (TOPK * M) // NW
CH = 64


def _pack_pair(lo_f32, hi_f32):
    lob = lax.shift_right_logical(
        lax.bitcast_convert_type(
            lo_f32.astype(jnp.bfloat16).astype(jnp.float32), jnp.int32), 16)
    hib = jnp.bitwise_and(
        lax.bitcast_convert_type(
            hi_f32.astype(jnp.bfloat16).astype(jnp.float32), jnp.int32),
        jnp.int32(-65536))
    return jnp.bitwise_or(hib, lob)


def _unpack_lo(xi):
    return lax.bitcast_convert_type(
        jnp.left_shift(xi, 16), jnp.float32).astype(jnp.bfloat16)


def _unpack_hi(xi):
    return lax.bitcast_convert_type(
        jnp.bitwise_and(xi, jnp.int32(-65536)),
        jnp.float32).astype(jnp.bfloat16)


def _meta_body(ids_ref, dest_ref, be_ref):
    ids = ids_ref[...]
    col = lax.broadcasted_iota(jnp.int32, (M, M), 1)
    row = lax.broadcasted_iota(jnp.int32, (M, M), 0)
    tri = (row <= col).astype(jnp.float32)
    occs, cexcls, counts = [], [], []
    for e in range(E):
        occ = (ids == e).astype(jnp.float32)
        c = lax.dot_general(occ, tri, (((1,), (0,)), ((), ())),
                            preferred_element_type=jnp.float32)
        c0_tot = c[0:1, M - 1:M]
        c = c + jnp.concatenate(
            [jnp.zeros((1, M), jnp.float32),
             jnp.broadcast_to(c0_tot, (1, M))], axis=0)
        occs.append(occ)
        cexcls.append(c - occ)
        counts.append(c[1:2, M - 1:M])
    dest = jnp.zeros((TOPK, M), jnp.float32)
    run = jnp.zeros((1, 1), jnp.float32)
    cumnb = []
    for e in range(E):
        dest = dest + occs[e] * (run * B + cexcls[e])
        nb = jnp.floor((counts[e] + (B - 1)) / B)
        run = run + nb
        cumnb.append(run)
    dest_ref[...] = dest.astype(jnp.int32)
    bidx = lax.broadcasted_iota(jnp.int32, (1, MAXNB), 1).astype(jnp.float32)
    be = jnp.zeros((1, MAXNB), jnp.float32)
    for e in range(E):
        be = be + (bidx >= jnp.broadcast_to(cumnb[e], (1, MAXNB))).astype(
            jnp.float32)
    be_ref[...] = be.astype(jnp.int32)


_meta = pl.pallas_call(
    _meta_body,
    out_shape=(jax.ShapeDtypeStruct((TOPK, M), jnp.int32),
               jax.ShapeDtypeStruct((1, MAXNB), jnp.int32)),
)


def _pack_body(a_ref, o_ref):
    a = a_ref[...]
    o_ref[...] = _pack_pair(a[:, :DH], a[:, DH:])


_pack = pl.pallas_call(
    _pack_body,
    grid=(M // B,),
    in_specs=[pl.BlockSpec((B, DM), lambda m: (m, 0))],
    out_specs=pl.BlockSpec((B, DH), lambda m: (m, 0)),
    out_shape=jax.ShapeDtypeStruct((M, DH), jnp.int32),
)


@functools.cache
def _sc_kernels():
    mesh = plsc.VectorSubcoreMesh(core_axis_name="c", subcore_axis_name="s")

    @functools.partial(
        pl.kernel,
        mesh=mesh,
        out_type=jax.ShapeDtypeStruct((P, DH), jnp.int32),
        scratch_types=[
            pltpu.VMEM((CH,), jnp.int32),
            pltpu.VMEM((CH, DH), jnp.int32),
            pltpu.SemaphoreType.DMA,
        ],
    )
    def dispatch(ap_hbm, destf_hbm, xp_hbm, idx_v, rows_v, sem):
        wid = lax.axis_index("s") * 2 + lax.axis_index("c")
        k = wid // (NW // TOPK)
        t0 = (wid % (NW // TOPK)) * TPW
        for ch in range(TPW // CH):
            base = t0 + ch * CH
            pltpu.sync_copy(destf_hbm.at[pl.ds(k * M + base, CH)], idx_v)
            pltpu.sync_copy(ap_hbm.at[pl.ds(base, CH), :], rows_v)
            pltpu.async_copy(rows_v, xp_hbm.at[idx_v], sem).wait()

    @functools.partial(
        pl.kernel,
        mesh=mesh,
        out_type=jax.ShapeDtypeStruct((TOPK * M, DH), jnp.int32),
        scratch_types=[
            pltpu.VMEM((CH,), jnp.int32),
            pltpu.VMEM((CH, DH), jnp.int32),
            pltpu.SemaphoreType.DMA,
        ],
    )
    def gather2(y2_hbm, destf_hbm, g_hbm, idx_v, rows_v, sem):
        wid = lax.axis_index("s") * 2 + lax.axis_index("c")
        k = wid // (NW // TOPK)
        t0 = (wid % (NW // TOPK)) * TPW
        for ch in range(TPW // CH):
            base = k * M + t0 + ch * CH
            pltpu.sync_copy(destf_hbm.at[pl.ds(base, CH)], idx_v)
            pltpu.async_copy(y2_hbm.at[idx_v], rows_v, sem).wait()
            pltpu.sync_copy(rows_v, g_hbm.at[pl.ds(base, CH), :])

    return dispatch, gather2


NS1 = 2
BN1 = DFF // NS1


def _mm1_body(be_ref, xp_ref, wgk0_ref, wgk1_ref, wuk0_ref, wuk1_ref,
              act_ref):
    b = pl.program_id(1)

    @pl.when(be_ref[b] < E)
    def _():
        xi = xp_ref[...]
        xlo = _unpack_lo(xi)
        xhi = _unpack_hi(xi)
        dn = (((1,), (1,)), ((), ()))
        hg = (lax.dot_general(xlo, wgk0_ref[0].astype(jnp.bfloat16), dn,
                              preferred_element_type=jnp.float32)
              + lax.dot_general(xhi, wgk1_ref[0].astype(jnp.bfloat16), dn,
                                preferred_element_type=jnp.float32))
        hu = (lax.dot_general(xlo, wuk0_ref[0].astype(jnp.bfloat16), dn,
                              preferred_element_type=jnp.float32)
              + lax.dot_general(xhi, wuk1_ref[0].astype(jnp.bfloat16), dn,
                                preferred_element_type=jnp.float32))
        actf = hg * jax.nn.sigmoid(hg) * hu
        act_ref[...] = _pack_pair(actf[:, :BN1 // 2], actf[:, BN1 // 2:])


def _mm1_gridspec():
    return pltpu.PrefetchScalarGridSpec(
        num_scalar_prefetch=1,
        grid=(NS1, MAXNB),
        in_specs=[
            pl.BlockSpec((B, DH),
                         lambda n, b, be: (jnp.where(be[b] < E, b, 0), 0)),
            pl.BlockSpec((1, BN1, DH),
                         lambda n, b, be: (jnp.minimum(be[b], E - 1), n, 0)),
            pl.BlockSpec((1, BN1, DH),
                         lambda n, b, be: (jnp.minimum(be[b], E - 1), n, 1)),
            pl.BlockSpec((1, BN1, DH),
                         lambda n, b, be: (jnp.minimum(be[b], E - 1), n + NS1,
                                           0)),
            pl.BlockSpec((1, BN1, DH),
                         lambda n, b, be: (jnp.minimum(be[b], E - 1), n + NS1,
                                           1)),
        ],
        out_specs=pl.BlockSpec(
            (B, BN1 // 2),
            lambda n, b, be: (jnp.where(be[b] < E, b, MAXNB - 1), n)),
    )


_mm1 = pl.pallas_call(
    _mm1_body,
    grid_spec=_mm1_gridspec(),
    out_shape=jax.ShapeDtypeStruct((P, DFF // 2), jnp.int32),
)


def _mm2_body(be_ref, act_ref, w0_ref, w1_ref, w2_ref, w3_ref, y2_ref):
    b = pl.program_id(0)
    Q = BN1 // 2

    @pl.when(be_ref[b] < E)
    def _():
        ai = act_ref[...]
        dn = (((1,), (1,)), ((), ()))
        ws = [w0_ref, w1_ref, w2_ref, w3_ref]
        y = None
        for n in range(NS1):
            blk = ai[:, n * Q:(n + 1) * Q]
            for half, unpack in ((0, _unpack_lo), (1, _unpack_hi)):
                w = ws[2 * n + half][0].astype(jnp.bfloat16)
                part = lax.dot_general(unpack(blk), w, dn,
                                       preferred_element_type=jnp.float32)
                y = part if y is None else y + part
        y2_ref[...] = _pack_pair(y[:, :DH], y[:, DH:])


def _mm2_gridspec():
    return pltpu.PrefetchScalarGridSpec(
        num_scalar_prefetch=1,
        grid=(MAXNB,),
        in_specs=[
            pl.BlockSpec((B, DFF // 2),
                         lambda b, be: (jnp.where(be[b] < E, b, 0), 0)),
        ] + [
            pl.BlockSpec((1, DM, DFF // 4),
                         lambda b, be, q=q: (jnp.minimum(be[b], E - 1), 0, q))
            for q in range(4)
        ],
        out_specs=pl.BlockSpec(
            (B, DH), lambda b, be: (jnp.where(be[b] < E, b, MAXNB - 1), 0)),
    )


_mm2 = pl.pallas_call(
    _mm2_body,
    grid_spec=_mm2_gridspec(),
    out_shape=jax.ShapeDtypeStruct((P, DH), jnp.int32),
)


def _comb_body(g_ref, tw_ref, out_ref):
    g = g_ref[...]
    tw = tw_ref[...]
    w0 = tw[:, 0:1]
    w1 = tw[:, 1:2]
    lo = (_unpack_lo(g[0]).astype(jnp.float32) * w0
          + _unpack_lo(g[1]).astype(jnp.float32) * w1)
    hi = (_unpack_hi(g[0]).astype(jnp.float32) * w0
          + _unpack_hi(g[1]).astype(jnp.float32) * w1)
    out_ref[...] = jnp.concatenate([lo, hi], axis=1)


_BM = 256
_comb = pl.pallas_call(
    _comb_body,
    grid=(M // _BM,),
    in_specs=[
        pl.BlockSpec((TOPK, _BM, DH), lambda m: (0, m, 0)),
        pl.BlockSpec((_BM, TOPK), lambda m: (m, 0)),
    ],
    out_specs=pl.BlockSpec((_BM, DM), lambda m: (m, 0)),
    out_shape=jax.ShapeDtypeStruct((M, DM), jnp.float32),
)


def kernel(a1, w1, w2, topk_weights, topk_ids):
    dispatch, gather2 = _sc_kernels()
    dest, be = _meta(topk_ids.T)
    destf = dest.reshape(TOPK * M)
    be_arr = be.reshape(MAXNB)
    ap = _pack(a1)
    xp = dispatch(ap, destf)
    act = _mm1(be_arr, xp, w1, w1, w1, w1)
    y2 = _mm2(be_arr, act, w2, w2, w2, w2)
    g = gather2(y2, destf)
    return _comb(g.reshape(TOPK, M, DH), topk_weights)
